# baseline (speedup 1.0000x reference)
import functools

import jax
import jax.numpy as jnp
from jax import lax
from jax.experimental import pallas as pl
from jax.experimental.pallas import tpu as pltpu
from jax.experimental.pallas import tpu_sc as plsc

_R = 128
_C = 32768
_G = 256
_CR = 8
_CC = 2048
_NSB = 4
_NCH = 8

_mesh = plsc.VectorSubcoreMesh(core_axis_name="c", subcore_axis_name="s")


@functools.partial(
    pl.kernel,
    mesh=_mesh,
    out_type=jax.ShapeDtypeStruct((_R, _C), jnp.float32),
    scratch_types=[
        pltpu.VMEM((_CR, _CC), jnp.float32),
        pltpu.VMEM((_CR, _CC), jnp.float32),
    ],
    compiler_params=pltpu.CompilerParams(needs_layout_passes=False),
)
def _sc_kernel(x_hbm, out_hbm, inbuf, outbuf):
    wid = lax.axis_index("s") * 2 + lax.axis_index("c")
    r0 = (wid % 16) * _CR
    c0 = (wid // 16) * (_NCH * _CC)
    lane = lax.iota(jnp.int32, 16)
    zeros = jnp.zeros((16,), jnp.float32)
    ones = jnp.ones((16,), jnp.float32)

    row_idx = jnp.bitwise_and(lane, 7)
    colb = [(jnp.right_shift(lane, 3) + 2 * b) * _G for b in range(_NSB)]

    def zero_body(i, _):
        for r in range(_CR):
            outbuf[r, pl.ds(i * 16, 16)] = zeros
        return _

    lax.fori_loop(0, _CC // 16, zero_body, None)

    def scan_elems(e, carry):
        curs, idxs = carry
        new_curs, new_idxs = [], []
        for b in range(_NSB):
            v = plsc.load_gather(inbuf, [row_idx, colb[b] + e])
            upd = v > curs[b]
            new_curs.append(jnp.where(upd, v, curs[b]))
            new_idxs.append(jnp.where(upd, e, idxs[b]))
        return tuple(new_curs), tuple(new_idxs)

    for c in range(_NCH):
        cc = c0 + c * _CC
        pltpu.sync_copy(x_hbm.at[pl.ds(r0, _CR), pl.ds(cc, _CC)], inbuf)
        curs0 = tuple(plsc.load_gather(inbuf, [row_idx, colb[b]])
                      for b in range(_NSB))
        idxs0 = tuple(jnp.zeros((16,), jnp.int32) for _ in range(_NSB))
        curs, idxs = lax.fori_loop(1, _G, scan_elems, (curs0, idxs0))
        onecols = [colb[b] + idxs[b] for b in range(_NSB)]
        for b in range(_NSB):
            plsc.store_scatter(outbuf, [row_idx, onecols[b]], ones)
        pltpu.sync_copy(outbuf, out_hbm.at[pl.ds(r0, _CR), pl.ds(cc, _CC)])
        for b in range(_NSB):
            plsc.store_scatter(outbuf, [row_idx, onecols[b]], zeros)


def kernel(x):
    return _sc_kernel(x)


# speedup vs baseline: 1.2847x; 1.2847x over previous
import functools

import jax
import jax.numpy as jnp
from jax import lax
from jax.experimental import pallas as pl
from jax.experimental.pallas import tpu as pltpu
from jax.experimental.pallas import tpu_sc as plsc

_R = 128
_C = 32768
_G = 256
_CR = 8
_CC = 2048
_NSB = 4
_NCH = 8

_mesh = plsc.VectorSubcoreMesh(core_axis_name="c", subcore_axis_name="s")


@functools.partial(
    pl.kernel,
    mesh=_mesh,
    out_type=jax.ShapeDtypeStruct((_R, _C), jnp.float32),
    scratch_types=[
        pltpu.VMEM((_CR, _CC), jnp.float32),
        pltpu.VMEM((_CR, _CC), jnp.float32),
        pltpu.VMEM((_CR, _CC), jnp.float32),
        pltpu.VMEM((_CR, _CC), jnp.float32),
        pltpu.SemaphoreType.DMA,
        pltpu.SemaphoreType.DMA,
        pltpu.SemaphoreType.DMA,
        pltpu.SemaphoreType.DMA,
    ],
    compiler_params=pltpu.CompilerParams(needs_layout_passes=False),
)
def _sc_kernel(x_hbm, out_hbm, in0, in1, ou0, ou1, si0, si1, so0, so1):
    ins, outs = [in0, in1], [ou0, ou1]
    isems, osems = [si0, si1], [so0, so1]
    wid = lax.axis_index("s") * 2 + lax.axis_index("c")
    r0 = (wid % 16) * _CR
    c0 = (wid // 16) * (_NCH * _CC)
    lane = lax.iota(jnp.int32, 16)
    zeros = jnp.zeros((16,), jnp.float32)
    ones = jnp.ones((16,), jnp.float32)
    neginf = jnp.full((16,), -jnp.inf, jnp.float32)

    row_idx = jnp.bitwise_and(lane, 7)
    colb = [(jnp.right_shift(lane, 3) + 2 * b) * _G for b in range(_NSB)]

    def zero_body(i, _):
        for r in range(_CR):
            ou0[r, pl.ds(i * 16, 16)] = zeros
            ou1[r, pl.ds(i * 16, 16)] = zeros
        return _

    lax.fori_loop(0, _CC // 16, zero_body, None)

    def make_scan(buf):
        def scan_elems(e, carry):
            curs, idxs = carry
            new_curs, new_idxs = [], []
            for b in range(_NSB):
                v = plsc.load_gather(buf, [row_idx, colb[b] + e])
                upd = v > curs[b]
                new_curs.append(jnp.maximum(curs[b], v))
                new_idxs.append(jnp.where(upd, e, idxs[b]))
            return tuple(new_curs), tuple(new_idxs)
        return scan_elems

    def start_in(c, b):
        cc = c0 + c * _CC
        return pltpu.async_copy(
            x_hbm.at[pl.ds(r0, _CR), pl.ds(cc, _CC)], ins[b], isems[b])

    def start_out(c, b):
        cc = c0 + c * _CC
        return pltpu.async_copy(
            outs[b], out_hbm.at[pl.ds(r0, _CR), pl.ds(cc, _CC)], osems[b])

    in_h = {0: start_in(0, 0)}
    out_h = {}
    prev_ones = [None, None]
    init = (tuple(neginf for _ in range(_NSB)),
            tuple(jnp.zeros((16,), jnp.int32) for _ in range(_NSB)))
    for c in range(_NCH):
        b = c & 1
        if c + 1 < _NCH:
            in_h[c + 1] = start_in(c + 1, 1 - b)
        in_h[c].wait()
        if c >= 2:
            out_h[c - 2].wait()
            for oc in prev_ones[b]:
                plsc.store_scatter(outs[b], [row_idx, oc], zeros)
        _, idxs = lax.fori_loop(0, _G, make_scan(ins[b]), init, unroll=8)
        onecols = [colb[k] + idxs[k] for k in range(_NSB)]
        for oc in onecols:
            plsc.store_scatter(outs[b], [row_idx, oc], ones)
        out_h[c] = start_out(c, b)
        prev_ones[b] = onecols
    out_h[_NCH - 2].wait()
    out_h[_NCH - 1].wait()


def kernel(x):
    return _sc_kernel(x)


# speedup vs baseline: 6.8381x; 5.3227x over previous
import jax
import jax.numpy as jnp
from jax import lax
from jax.experimental import pallas as pl

_R = 128
_C = 32768
_G = 256
_BLOCK_C = 4096


def _body(x_ref, o_ref):
    iota = lax.broadcasted_iota(jnp.int32, (_R, _G), 1).astype(jnp.float32)
    for k in range(_BLOCK_C // _G):
        xb = x_ref[:, k * _G:(k + 1) * _G]
        m = jnp.max(xb, axis=1, keepdims=True)
        eq = xb == m
        imin = jnp.min(jnp.where(eq, iota, 512.0), axis=1, keepdims=True)
        o_ref[:, k * _G:(k + 1) * _G] = jnp.where(iota == imin, 1.0, 0.0)


def kernel(x):
    return pl.pallas_call(
        _body,
        grid=(_C // _BLOCK_C,),
        in_specs=[pl.BlockSpec((_R, _BLOCK_C), lambda j: (0, j))],
        out_specs=pl.BlockSpec((_R, _BLOCK_C), lambda j: (0, j)),
        out_shape=jax.ShapeDtypeStruct((_R, _C), jnp.float32),
    )(x)
